# baseline (speedup 1.0000x reference)
import jax
import jax.numpy as jnp
from jax.experimental import pallas as pl

_B, _H = 16384, 200
_FLAT_ROWS = _B * _H // 256
_ROW_BLK = 256


def _body(xx_ref, xy_ref, rx_ref, ry_ref):
    scale = jnp.float32(2.0 / 3.0)
    rx_ref[...] = xx_ref[...].astype(jnp.float32) * scale - 1.0
    ry_ref[...] = xy_ref[...].astype(jnp.float32) * scale - 1.0


def kernel(x_x, x_y):
    xf = x_x.reshape(_FLAT_ROWS, 256)
    yf = x_y.reshape(_FLAT_ROWS, 256)
    grid = (_FLAT_ROWS // _ROW_BLK,)
    spec = pl.BlockSpec((_ROW_BLK, 256), lambda i: (i, 0))
    rx, ry = pl.pallas_call(
        _body,
        grid=grid,
        in_specs=[spec, spec],
        out_specs=[spec, spec],
        out_shape=[jax.ShapeDtypeStruct((_FLAT_ROWS, 256), jnp.float32)] * 2,
    )(xf, yf)
    out = jnp.stack((rx.reshape(_B, _H), ry.reshape(_B, _H)), axis=-1)
    return out.astype(jnp.complex64)
